# Initial kernel scaffold; baseline (speedup 1.0000x reference)
import functools

import jax
import jax.numpy as jnp
import numpy as np
from jax import lax
from jax.experimental import pallas as pl
from jax.experimental.pallas import tpu as pltpu
from jax.experimental.pallas import tpu_sc as plsc

N_NODES = 10000
N_EDGES = 320000
HIDDEN = 128
NUM_RBF = 16
CUTOFF_UPPER = 5.0

NC = 2
NS = 16
NW = NC * NS
EDGES_PER_W = N_EDGES // NW
EB = 80
NBLK = EDGES_PER_W // EB
ROWS_PER_TILE = N_NODES // NS


def _silu(v):
    return v * (1.0 / (1.0 + jnp.exp(-v)))


def _filter_body(ea_ref, ew_ref, w1t_ref, b1_ref, w2t_ref, b2_ref, o_ref):
    ea = ea_ref[...]
    h = jnp.dot(ea, w1t_ref[...], preferred_element_type=jnp.float32)
    h = h + b1_ref[...]
    h = _silu(h)
    w = jnp.dot(h, w2t_ref[...], preferred_element_type=jnp.float32)
    w = w + b2_ref[...]
    ew = ew_ref[...]
    c = 0.5 * (jnp.cos(ew * (np.pi / CUTOFF_UPPER)) + 1.0)
    c = c * (ew < CUTOFF_UPPER).astype(jnp.float32)
    o_ref[...] = w * c


def _filter_w(edge_attr, ew2, w1t, b1, w2t, b2):
    BE = 4000
    grid = N_EDGES // BE
    return pl.pallas_call(
        _filter_body,
        grid=(grid,),
        in_specs=[
            pl.BlockSpec((BE, NUM_RBF), lambda i: (i, 0)),
            pl.BlockSpec((BE, 1), lambda i: (i, 0)),
            pl.BlockSpec((NUM_RBF, HIDDEN), lambda i: (0, 0)),
            pl.BlockSpec((1, HIDDEN), lambda i: (0, 0)),
            pl.BlockSpec((HIDDEN, HIDDEN), lambda i: (0, 0)),
            pl.BlockSpec((1, HIDDEN), lambda i: (0, 0)),
        ],
        out_specs=pl.BlockSpec((BE, HIDDEN), lambda i: (i, 0)),
        out_shape=jax.ShapeDtypeStruct((N_EDGES, HIDDEN), jnp.float32),
    )(edge_attr, ew2, w1t, b1, w2t, b2)


def _xh_body(x_ref, wt_ref, o_ref):
    o_ref[...] = jnp.dot(x_ref[...], wt_ref[...],
                         preferred_element_type=jnp.float32)


def _xh(x, lin1t):
    BN = 2000
    grid = N_NODES // BN
    return pl.pallas_call(
        _xh_body,
        grid=(grid,),
        in_specs=[
            pl.BlockSpec((BN, HIDDEN), lambda i: (i, 0)),
            pl.BlockSpec((HIDDEN, HIDDEN), lambda i: (0, 0)),
        ],
        out_specs=pl.BlockSpec((BN, HIDDEN), lambda i: (i, 0)),
        out_shape=jax.ShapeDtypeStruct((N_NODES, HIDDEN), jnp.float32),
    )(x, lin1t)


def _sc_body(xh_hbm, w_hbm, src_hbm, dst_hbm, out_hbm,
             acc, src_v, dst_v, xj, wv, msg, zbuf, sem_g, sem_w):
    c = lax.axis_index("c")
    s = lax.axis_index("s")
    wid = c * NS + s

    @pl.loop(0, 125)
    def _zb(j):
        for k in range(9):
            zbuf[j, k] = jnp.zeros((16,), jnp.float32)

    for b in range(ROWS_PER_TILE // 125):
        pltpu.sync_copy(zbuf, acc.at[pl.ds(s * ROWS_PER_TILE + b * 125, 125)])

    @pl.loop(0, EB)
    def _ones(j):
        msg[j, 8] = jnp.ones((16,), jnp.float32)

    plsc.subcore_barrier()

    base = wid * EDGES_PER_W

    @pl.loop(0, NBLK)
    def _blk(i):
        off = base + i * EB
        pltpu.sync_copy(src_hbm.at[pl.ds(off, EB)], src_v)
        pltpu.sync_copy(dst_hbm.at[pl.ds(off, EB)], dst_v)
        gcp = pltpu.async_copy(xh_hbm.at[src_v], xj, sem_g)
        wcp = pltpu.async_copy(w_hbm.at[pl.ds(off, EB)], wv, sem_w)
        gcp.wait()
        wcp.wait()

        @pl.loop(0, EB)
        def _mul(j):
            for k in range(8):
                msg[j, k] = xj[j, k] * wv[j, k]

        pltpu.sync_copy(msg, acc.at[dst_v], add=True)

    plsc.subcore_barrier()
    pltpu.sync_copy(acc.at[pl.ds(s * ROWS_PER_TILE, ROWS_PER_TILE)],
                    out_hbm.at[c, pl.ds(s * ROWS_PER_TILE, ROWS_PER_TILE)])


def _sc_msg(xh3, w3, src, dst):
    mesh = plsc.VectorSubcoreMesh(core_axis_name="c", subcore_axis_name="s",
                                  num_cores=NC, num_subcores=NS)
    fn = pl.kernel(
        _sc_body,
        out_type=jax.ShapeDtypeStruct((NC, N_NODES, 9, 16), jnp.float32),
        mesh=mesh,
        scratch_types=[
            pltpu.VMEM_SHARED((N_NODES, 9, 16), jnp.float32),
            pltpu.VMEM((EB,), jnp.int32),
            pltpu.VMEM((EB,), jnp.int32),
            pltpu.VMEM((EB, 8, 16), jnp.float32),
            pltpu.VMEM((EB, 8, 16), jnp.float32),
            pltpu.VMEM((EB, 9, 16), jnp.float32),
            pltpu.VMEM((125, 9, 16), jnp.float32),
            pltpu.SemaphoreType.DMA,
            pltpu.SemaphoreType.DMA,
        ],
    )
    return fn(xh3, w3, src, dst)


def _tail_body(agg_ref, cnt_ref, l2t_ref, l2b_ref, lt_ref, lb_ref, o_ref):
    ssum = agg_ref[0] + agg_ref[1]
    cnt = cnt_ref[0] + cnt_ref[1]
    denom = jnp.where(cnt > 0, cnt, 1.0)
    agg = ssum / denom
    t = jnp.dot(agg, l2t_ref[...], preferred_element_type=jnp.float32)
    t = t + l2b_ref[...]
    t = _silu(t)
    o = jnp.dot(t, lt_ref[...], preferred_element_type=jnp.float32)
    o_ref[...] = o + lb_ref[...]


def _tail(agg2, cnt2, lin2t, lin2_b2, lint, lin_b2):
    BN = 2000
    grid = N_NODES // BN
    return pl.pallas_call(
        _tail_body,
        grid=(grid,),
        in_specs=[
            pl.BlockSpec((NC, BN, HIDDEN), lambda i: (0, i, 0)),
            pl.BlockSpec((NC, BN, 1), lambda i: (0, i, 0)),
            pl.BlockSpec((HIDDEN, HIDDEN), lambda i: (0, 0)),
            pl.BlockSpec((1, HIDDEN), lambda i: (0, 0)),
            pl.BlockSpec((HIDDEN, HIDDEN), lambda i: (0, 0)),
            pl.BlockSpec((1, HIDDEN), lambda i: (0, 0)),
        ],
        out_specs=pl.BlockSpec((BN, HIDDEN), lambda i: (i, 0)),
        out_shape=jax.ShapeDtypeStruct((N_NODES, HIDDEN), jnp.float32),
    )(agg2, cnt2, lin2t, lin2_b2, lint, lin_b2)


def kernel(x, edge_index, edge_weight, edge_attr,
           mlp_w1, mlp_b1, mlp_w2, mlp_b2,
           lin1_w, lin2_w, lin2_b, lin_w, lin_b):
    src = edge_index[0].astype(jnp.int32)
    dst = edge_index[1].astype(jnp.int32)
    ew2 = edge_weight.reshape(N_EDGES, 1)

    w1t = mlp_w1.T
    w2t = mlp_w2.T
    lin1t = lin1_w.T
    lin2t = lin2_w.T
    lint = lin_w.T
    b1 = mlp_b1.reshape(1, HIDDEN)
    b2 = mlp_b2.reshape(1, HIDDEN)
    lin2_b2 = lin2_b.reshape(1, HIDDEN)
    lin_b2 = lin_b.reshape(1, HIDDEN)

    w = _filter_w(edge_attr, ew2, w1t, b1, w2t, b2)
    xh = _xh(x, lin1t)

    xh3 = xh.reshape(N_NODES, 8, 16)
    w3 = w.reshape(N_EDGES, 8, 16)
    out3 = _sc_msg(xh3, w3, src, dst)

    agg2 = out3[:, :, :8, :].reshape(NC, N_NODES, HIDDEN)
    cnt2 = out3[:, :, 8, 0].reshape(NC, N_NODES, 1)
    return _tail(agg2, cnt2, lin2t, lin2_b2, lint, lin_b2)


# speedup vs baseline: 1.8212x; 1.8212x over previous
import jax
import jax.numpy as jnp
import numpy as np
from jax import lax
from jax.experimental import pallas as pl
from jax.experimental.pallas import tpu as pltpu
from jax.experimental.pallas import tpu_sc as plsc

N_NODES = 10000
N_PAD = 10240
N_EDGES = 320000
HIDDEN = 128
NUM_RBF = 16
CUTOFF_UPPER = 5.0

NC = 2
NS = 16
NW = NC * NS
EDGES_PER_W = N_EDGES // NW
EB = 80
NBLK = EDGES_PER_W // EB
ROWS_PER_TILE = N_PAD // NS


def _silu(v):
    return v * (1.0 / (1.0 + jnp.exp(-v)))


def _filter_body(ea_ref, ew_ref, w1t_ref, b1_ref, w2t_ref, b2_ref, o_ref):
    ea = ea_ref[...]
    h = jnp.dot(ea, w1t_ref[...], preferred_element_type=jnp.float32)
    h = h + b1_ref[...]
    h = _silu(h)
    w = jnp.dot(h, w2t_ref[...], preferred_element_type=jnp.float32)
    w = w + b2_ref[...]
    ew = ew_ref[...]
    c = 0.5 * (jnp.cos(ew * (np.pi / CUTOFF_UPPER)) + 1.0)
    c = c * (ew < CUTOFF_UPPER).astype(jnp.float32)
    o_ref[...] = w * c


def _filter_w(edge_attr, ew2, w1t, b1, w2t, b2):
    BE = 4000
    grid = N_EDGES // BE
    return pl.pallas_call(
        _filter_body,
        grid=(grid,),
        in_specs=[
            pl.BlockSpec((BE, NUM_RBF), lambda i: (i, 0)),
            pl.BlockSpec((BE, 1), lambda i: (i, 0)),
            pl.BlockSpec((NUM_RBF, HIDDEN), lambda i: (0, 0)),
            pl.BlockSpec((1, HIDDEN), lambda i: (0, 0)),
            pl.BlockSpec((HIDDEN, HIDDEN), lambda i: (0, 0)),
            pl.BlockSpec((1, HIDDEN), lambda i: (0, 0)),
        ],
        out_specs=pl.BlockSpec((BE, HIDDEN), lambda i: (i, 0)),
        out_shape=jax.ShapeDtypeStruct((N_EDGES, HIDDEN), jnp.float32),
    )(edge_attr, ew2, w1t, b1, w2t, b2)


def _xh_body(x_ref, wt_ref, o_ref):
    o_ref[...] = jnp.dot(x_ref[...], wt_ref[...],
                         preferred_element_type=jnp.float32)


def _xh(x, lin1t):
    BN = 2000
    grid = N_NODES // BN
    return pl.pallas_call(
        _xh_body,
        grid=(grid,),
        in_specs=[
            pl.BlockSpec((BN, HIDDEN), lambda i: (i, 0)),
            pl.BlockSpec((HIDDEN, HIDDEN), lambda i: (0, 0)),
        ],
        out_specs=pl.BlockSpec((BN, HIDDEN), lambda i: (i, 0)),
        out_shape=jax.ShapeDtypeStruct((N_NODES, HIDDEN), jnp.float32),
    )(x, lin1t)


def _sc_body(xh_hbm, w_hbm, src_hbm, dst_hbm, out_hbm, cnt_hbm,
             acc, cnt, src_v, dst_v, xj, wv, msg, ones_v, zbuf, zcnt,
             sem_g, sem_w):
    c = lax.axis_index("c")
    s = lax.axis_index("s")
    wid = c * NS + s

    @pl.loop(0, 128)
    def _zb(j):
        for k in range(8):
            zbuf[j, pl.ds(k * 16, 16)] = jnp.zeros((16,), jnp.float32)

    @pl.loop(0, ROWS_PER_TILE // 16)
    def _zc(j):
        zcnt[pl.ds(j * 16, 16)] = jnp.zeros((16,), jnp.float32)

    @pl.loop(0, EB // 16)
    def _on(j):
        ones_v[pl.ds(j * 16, 16)] = jnp.ones((16,), jnp.float32)

    for b in range(ROWS_PER_TILE // 128):
        pltpu.sync_copy(zbuf, acc.at[pl.ds(s * ROWS_PER_TILE + b * 128, 128)])
    pltpu.sync_copy(zcnt, cnt.at[pl.ds(s * ROWS_PER_TILE, ROWS_PER_TILE)])

    plsc.subcore_barrier()

    base = wid * EDGES_PER_W

    @pl.loop(0, NBLK)
    def _blk(i):
        off = base + i * EB
        pltpu.sync_copy(src_hbm.at[pl.ds(off, EB)], src_v)
        pltpu.sync_copy(dst_hbm.at[pl.ds(off, EB)], dst_v)
        gcp = pltpu.async_copy(xh_hbm.at[src_v], xj, sem_g)
        wcp = pltpu.async_copy(w_hbm.at[pl.ds(off, EB)], wv, sem_w)
        gcp.wait()
        wcp.wait()

        @pl.loop(0, EB)
        def _mul(j):
            for k in range(8):
                sl = pl.ds(k * 16, 16)
                msg[j, sl] = xj[j, sl] * wv[j, sl]

        pltpu.sync_copy(msg, acc.at[dst_v], add=True)
        pltpu.sync_copy(ones_v, cnt.at[dst_v], add=True)

    plsc.subcore_barrier()
    pltpu.sync_copy(acc.at[pl.ds(s * ROWS_PER_TILE, ROWS_PER_TILE)],
                    out_hbm.at[c, pl.ds(s * ROWS_PER_TILE, ROWS_PER_TILE)])
    pltpu.sync_copy(cnt.at[pl.ds(s * ROWS_PER_TILE, ROWS_PER_TILE)],
                    cnt_hbm.at[c, pl.ds(s * ROWS_PER_TILE, ROWS_PER_TILE)])


def _sc_msg(xh, w, src, dst):
    mesh = plsc.VectorSubcoreMesh(core_axis_name="c", subcore_axis_name="s",
                                  num_cores=NC, num_subcores=NS)
    fn = pl.kernel(
        _sc_body,
        out_type=[
            jax.ShapeDtypeStruct((NC, N_PAD, HIDDEN), jnp.float32),
            jax.ShapeDtypeStruct((NC, N_PAD), jnp.float32),
        ],
        mesh=mesh,
        scratch_types=[
            pltpu.VMEM_SHARED((N_PAD, HIDDEN), jnp.float32),
            pltpu.VMEM_SHARED((N_PAD,), jnp.float32),
            pltpu.VMEM((EB,), jnp.int32),
            pltpu.VMEM((EB,), jnp.int32),
            pltpu.VMEM((EB, HIDDEN), jnp.float32),
            pltpu.VMEM((EB, HIDDEN), jnp.float32),
            pltpu.VMEM((EB, HIDDEN), jnp.float32),
            pltpu.VMEM((EB,), jnp.float32),
            pltpu.VMEM((128, HIDDEN), jnp.float32),
            pltpu.VMEM((ROWS_PER_TILE,), jnp.float32),
            pltpu.SemaphoreType.DMA,
            pltpu.SemaphoreType.DMA,
        ],
    )
    return fn(xh, w, src, dst)


def _tail_body(agg_ref, cnt_ref, l2t_ref, l2b_ref, lt_ref, lb_ref, o_ref):
    ssum = agg_ref[0] + agg_ref[1]
    cnt = cnt_ref[...]
    denom = jnp.where(cnt > 0, cnt, 1.0)
    agg = ssum / denom
    t = jnp.dot(agg, l2t_ref[...], preferred_element_type=jnp.float32)
    t = t + l2b_ref[...]
    t = _silu(t)
    o = jnp.dot(t, lt_ref[...], preferred_element_type=jnp.float32)
    o_ref[...] = o + lb_ref[...]


def _tail(agg2, cnt1, lin2t, lin2_b2, lint, lin_b2):
    BN = 2000
    grid = N_NODES // BN
    return pl.pallas_call(
        _tail_body,
        grid=(grid,),
        in_specs=[
            pl.BlockSpec((NC, BN, HIDDEN), lambda i: (0, i, 0)),
            pl.BlockSpec((BN, 1), lambda i: (i, 0)),
            pl.BlockSpec((HIDDEN, HIDDEN), lambda i: (0, 0)),
            pl.BlockSpec((1, HIDDEN), lambda i: (0, 0)),
            pl.BlockSpec((HIDDEN, HIDDEN), lambda i: (0, 0)),
            pl.BlockSpec((1, HIDDEN), lambda i: (0, 0)),
        ],
        out_specs=pl.BlockSpec((BN, HIDDEN), lambda i: (i, 0)),
        out_shape=jax.ShapeDtypeStruct((N_NODES, HIDDEN), jnp.float32),
    )(agg2, cnt1, lin2t, lin2_b2, lint, lin_b2)


def kernel(x, edge_index, edge_weight, edge_attr,
           mlp_w1, mlp_b1, mlp_w2, mlp_b2,
           lin1_w, lin2_w, lin2_b, lin_w, lin_b):
    src = edge_index[0].astype(jnp.int32)
    dst = edge_index[1].astype(jnp.int32)
    ew2 = edge_weight.reshape(N_EDGES, 1)

    w1t = mlp_w1.T
    w2t = mlp_w2.T
    lin1t = lin1_w.T
    lin2t = lin2_w.T
    lint = lin_w.T
    b1 = mlp_b1.reshape(1, HIDDEN)
    b2 = mlp_b2.reshape(1, HIDDEN)
    lin2_b2 = lin2_b.reshape(1, HIDDEN)
    lin_b2 = lin_b.reshape(1, HIDDEN)

    w = _filter_w(edge_attr, ew2, w1t, b1, w2t, b2)
    xh = _xh(x, lin1t)

    agg2, cnt2 = _sc_msg(xh, w, src, dst)

    agg2 = agg2[:, :N_NODES, :]
    cnt1 = (cnt2[0] + cnt2[1])[:N_NODES].reshape(N_NODES, 1)
    return _tail(agg2, cnt1, lin2t, lin2_b2, lint, lin_b2)


# speedup vs baseline: 3.0575x; 1.6788x over previous
import jax
import jax.numpy as jnp
import numpy as np
from jax import lax
from jax.experimental import pallas as pl
from jax.experimental.pallas import tpu as pltpu
from jax.experimental.pallas import tpu_sc as plsc

N_NODES = 10000
N_PAD = 10240
N_EDGES = 320000
HIDDEN = 128
NUM_RBF = 16
CUTOFF_UPPER = 5.0

NC = 2
NS = 16
NW = NC * NS
EDGES_PER_W = N_EDGES // NW
EB = 80
NBLK = EDGES_PER_W // EB
ROWS_PER_TILE = N_PAD // NS


def _silu(v):
    return v * (1.0 / (1.0 + jnp.exp(-v)))


def _filter_body(ea_ref, w1t_ref, b1_ref, w2t_ref, b2_ref, o_ref):
    ea = ea_ref[...]
    h = jnp.dot(ea, w1t_ref[...], preferred_element_type=jnp.float32)
    h = h + b1_ref[...]
    h = _silu(h)
    w = jnp.dot(h, w2t_ref[...], preferred_element_type=jnp.float32)
    o_ref[...] = w + b2_ref[...]


def _filter_w(edge_attr, w1t, b1, w2t, b2):
    BE = 4000
    grid = N_EDGES // BE
    return pl.pallas_call(
        _filter_body,
        grid=(grid,),
        in_specs=[
            pl.BlockSpec((BE, NUM_RBF), lambda i: (i, 0)),
            pl.BlockSpec((NUM_RBF, HIDDEN), lambda i: (0, 0)),
            pl.BlockSpec((1, HIDDEN), lambda i: (0, 0)),
            pl.BlockSpec((HIDDEN, HIDDEN), lambda i: (0, 0)),
            pl.BlockSpec((1, HIDDEN), lambda i: (0, 0)),
        ],
        out_specs=pl.BlockSpec((BE, HIDDEN), lambda i: (i, 0)),
        out_shape=jax.ShapeDtypeStruct((N_EDGES, HIDDEN), jnp.float32),
    )(edge_attr, w1t, b1, w2t, b2)


def _cutoff_body(ew_ref, c_ref):
    ew = ew_ref[...]
    c = 0.5 * (jnp.cos(ew * (np.pi / CUTOFF_UPPER)) + 1.0)
    c_ref[...] = c * (ew < CUTOFF_UPPER).astype(jnp.float32)


def _cutoff(edge_weight):
    return pl.pallas_call(
        _cutoff_body,
        out_shape=jax.ShapeDtypeStruct((N_EDGES,), jnp.float32),
    )(edge_weight)


def _xh_body(x_ref, wt_ref, o_ref):
    o_ref[...] = jnp.dot(x_ref[...], wt_ref[...],
                         preferred_element_type=jnp.float32)


def _xh(x, lin1t):
    BN = 2000
    grid = N_NODES // BN
    return pl.pallas_call(
        _xh_body,
        grid=(grid,),
        in_specs=[
            pl.BlockSpec((BN, HIDDEN), lambda i: (i, 0)),
            pl.BlockSpec((HIDDEN, HIDDEN), lambda i: (0, 0)),
        ],
        out_specs=pl.BlockSpec((BN, HIDDEN), lambda i: (i, 0)),
        out_shape=jax.ShapeDtypeStruct((N_NODES, HIDDEN), jnp.float32),
    )(x, lin1t)


def _sc_body(xh_hbm, w_hbm, c_hbm, src_hbm, dst_hbm, out_hbm, cnt_hbm,
             acc, cnt, src_v, dst_v, cv, xj, wv, msg, ones_v, zbuf, zcnt,
             sem_g, sem_w):
    c = lax.axis_index("c")
    s = lax.axis_index("s")
    wid = c * NS + s

    @pl.loop(0, 128)
    def _zb(j):
        for k in range(8):
            zbuf[j, pl.ds(k * 16, 16)] = jnp.zeros((16,), jnp.float32)

    @pl.loop(0, ROWS_PER_TILE // 16)
    def _zc(j):
        zcnt[pl.ds(j * 16, 16)] = jnp.zeros((16,), jnp.float32)

    @pl.loop(0, EB // 16)
    def _on(j):
        ones_v[pl.ds(j * 16, 16)] = jnp.ones((16,), jnp.float32)

    for b in range(ROWS_PER_TILE // 128):
        pltpu.sync_copy(zbuf, acc.at[pl.ds(s * ROWS_PER_TILE + b * 128, 128)])
    pltpu.sync_copy(zcnt, cnt.at[pl.ds(s * ROWS_PER_TILE, ROWS_PER_TILE)])

    plsc.subcore_barrier()

    base = wid * EDGES_PER_W

    @pl.loop(0, NBLK)
    def _blk(i):
        off = base + i * EB
        pltpu.sync_copy(src_hbm.at[pl.ds(off, EB)], src_v)
        pltpu.sync_copy(dst_hbm.at[pl.ds(off, EB)], dst_v)
        pltpu.sync_copy(c_hbm.at[pl.ds(off, EB)], cv.at[pl.ds(0, EB)])
        gcp = pltpu.async_copy(xh_hbm.at[src_v], xj, sem_g)
        wcp = pltpu.async_copy(w_hbm.at[pl.ds(off, EB)], wv, sem_w)
        gcp.wait()
        wcp.wait()

        @pl.loop(0, EB)
        def _mul(j):
            cj = cv[pl.ds(j, 16)][0]
            for k in range(8):
                sl = pl.ds(k * 16, 16)
                msg[j, sl] = xj[j, sl] * wv[j, sl] * cj

        pltpu.sync_copy(msg, acc.at[dst_v], add=True)
        pltpu.sync_copy(ones_v, cnt.at[dst_v], add=True)

    plsc.subcore_barrier()
    pltpu.sync_copy(acc.at[pl.ds(s * ROWS_PER_TILE, ROWS_PER_TILE)],
                    out_hbm.at[c, pl.ds(s * ROWS_PER_TILE, ROWS_PER_TILE)])
    pltpu.sync_copy(cnt.at[pl.ds(s * ROWS_PER_TILE, ROWS_PER_TILE)],
                    cnt_hbm.at[c, pl.ds(s * ROWS_PER_TILE, ROWS_PER_TILE)])


def _sc_msg(xh, w, cearr, src, dst):
    mesh = plsc.VectorSubcoreMesh(core_axis_name="c", subcore_axis_name="s",
                                  num_cores=NC, num_subcores=NS)
    fn = pl.kernel(
        _sc_body,
        out_type=[
            jax.ShapeDtypeStruct((NC, N_PAD, HIDDEN), jnp.float32),
            jax.ShapeDtypeStruct((NC, N_PAD), jnp.float32),
        ],
        mesh=mesh,
        scratch_types=[
            pltpu.VMEM_SHARED((N_PAD, HIDDEN), jnp.float32),
            pltpu.VMEM_SHARED((N_PAD,), jnp.float32),
            pltpu.VMEM((EB,), jnp.int32),
            pltpu.VMEM((EB,), jnp.int32),
            pltpu.VMEM((EB + 16,), jnp.float32),
            pltpu.VMEM((EB, HIDDEN), jnp.float32),
            pltpu.VMEM((EB, HIDDEN), jnp.float32),
            pltpu.VMEM((EB, HIDDEN), jnp.float32),
            pltpu.VMEM((EB,), jnp.float32),
            pltpu.VMEM((128, HIDDEN), jnp.float32),
            pltpu.VMEM((ROWS_PER_TILE,), jnp.float32),
            pltpu.SemaphoreType.DMA,
            pltpu.SemaphoreType.DMA,
        ],
    )
    return fn(xh, w, cearr, src, dst)


def _tail_body(agg_ref, cnt_ref, l2t_ref, l2b_ref, lt_ref, lb_ref, o_ref):
    ssum = agg_ref[0] + agg_ref[1]
    cnt = cnt_ref[...]
    denom = jnp.where(cnt > 0, cnt, 1.0)
    agg = ssum / denom
    t = jnp.dot(agg, l2t_ref[...], preferred_element_type=jnp.float32)
    t = t + l2b_ref[...]
    t = _silu(t)
    o = jnp.dot(t, lt_ref[...], preferred_element_type=jnp.float32)
    o_ref[...] = o + lb_ref[...]


def _tail(agg2, cnt1, lin2t, lin2_b2, lint, lin_b2):
    BN = 2000
    grid = N_NODES // BN
    return pl.pallas_call(
        _tail_body,
        grid=(grid,),
        in_specs=[
            pl.BlockSpec((NC, BN, HIDDEN), lambda i: (0, i, 0)),
            pl.BlockSpec((BN, 1), lambda i: (i, 0)),
            pl.BlockSpec((HIDDEN, HIDDEN), lambda i: (0, 0)),
            pl.BlockSpec((1, HIDDEN), lambda i: (0, 0)),
            pl.BlockSpec((HIDDEN, HIDDEN), lambda i: (0, 0)),
            pl.BlockSpec((1, HIDDEN), lambda i: (0, 0)),
        ],
        out_specs=pl.BlockSpec((BN, HIDDEN), lambda i: (i, 0)),
        out_shape=jax.ShapeDtypeStruct((N_NODES, HIDDEN), jnp.float32),
    )(agg2, cnt1, lin2t, lin2_b2, lint, lin_b2)


def kernel(x, edge_index, edge_weight, edge_attr,
           mlp_w1, mlp_b1, mlp_w2, mlp_b2,
           lin1_w, lin2_w, lin2_b, lin_w, lin_b):
    src = edge_index[0].astype(jnp.int32)
    dst = edge_index[1].astype(jnp.int32)

    w1t = mlp_w1.T
    w2t = mlp_w2.T
    lin1t = lin1_w.T
    lin2t = lin2_w.T
    lint = lin_w.T
    b1 = mlp_b1.reshape(1, HIDDEN)
    b2 = mlp_b2.reshape(1, HIDDEN)
    lin2_b2 = lin2_b.reshape(1, HIDDEN)
    lin_b2 = lin_b.reshape(1, HIDDEN)

    w = _filter_w(edge_attr, w1t, b1, w2t, b2)
    cearr = _cutoff(edge_weight)
    xh = _xh(x, lin1t)

    agg2, cnt2 = _sc_msg(xh, w, cearr, src, dst)

    agg2 = agg2[:, :N_NODES, :]
    cnt1 = (cnt2[0] + cnt2[1])[:N_NODES].reshape(N_NODES, 1)
    return _tail(agg2, cnt1, lin2t, lin2_b2, lint, lin_b2)


# speedup vs baseline: 4.4271x; 1.4480x over previous
import jax
import jax.numpy as jnp
import numpy as np
from jax import lax
from jax.experimental import pallas as pl
from jax.experimental.pallas import tpu as pltpu
from jax.experimental.pallas import tpu_sc as plsc

N_NODES = 10000
N_PAD = 10240
N_EDGES = 320000
HIDDEN = 128
NUM_RBF = 16
CUTOFF_UPPER = 5.0

NC = 2
NS = 16
NW = NC * NS
EDGES_PER_W = N_EDGES // NW
EB = 80
NBLK = EDGES_PER_W // EB
ROWS_PER_TILE = N_PAD // NS


def _silu(v):
    return v * (1.0 / (1.0 + jnp.exp(-v)))


def _filter_body(eat_ref, w1t_ref, b1_ref, w2t_ref, b2_ref, o_ref):
    eat = eat_ref[...]
    h = lax.dot_general(eat, w1t_ref[...], (((0,), (0,)), ((), ())),
                        preferred_element_type=jnp.float32)
    h = h + b1_ref[...]
    h = _silu(h)
    w = jnp.dot(h, w2t_ref[...], preferred_element_type=jnp.float32)
    o_ref[...] = w + b2_ref[...]


def _filter_w(eat, w1t, b1, w2t, b2):
    BE = 3200
    grid = N_EDGES // BE
    return pl.pallas_call(
        _filter_body,
        grid=(grid,),
        in_specs=[
            pl.BlockSpec((NUM_RBF, BE), lambda i: (0, i)),
            pl.BlockSpec((NUM_RBF, HIDDEN), lambda i: (0, 0)),
            pl.BlockSpec((1, HIDDEN), lambda i: (0, 0)),
            pl.BlockSpec((HIDDEN, HIDDEN), lambda i: (0, 0)),
            pl.BlockSpec((1, HIDDEN), lambda i: (0, 0)),
        ],
        out_specs=pl.BlockSpec((BE, HIDDEN), lambda i: (i, 0)),
        out_shape=jax.ShapeDtypeStruct((N_EDGES, HIDDEN), jnp.float32),
    )(eat, w1t, b1, w2t, b2)


def _cutoff_body(ew_ref, c_ref):
    ew = ew_ref[...]
    c = 0.5 * (jnp.cos(ew * (np.pi / CUTOFF_UPPER)) + 1.0)
    c_ref[...] = c * (ew < CUTOFF_UPPER).astype(jnp.float32)


def _cutoff(edge_weight):
    return pl.pallas_call(
        _cutoff_body,
        out_shape=jax.ShapeDtypeStruct((N_EDGES,), jnp.float32),
    )(edge_weight)


def _xh_body(x_ref, wt_ref, o_ref):
    o_ref[...] = jnp.dot(x_ref[...], wt_ref[...],
                         preferred_element_type=jnp.float32)


def _xh(x, lin1t):
    BN = 2000
    grid = N_NODES // BN
    return pl.pallas_call(
        _xh_body,
        grid=(grid,),
        in_specs=[
            pl.BlockSpec((BN, HIDDEN), lambda i: (i, 0)),
            pl.BlockSpec((HIDDEN, HIDDEN), lambda i: (0, 0)),
        ],
        out_specs=pl.BlockSpec((BN, HIDDEN), lambda i: (i, 0)),
        out_shape=jax.ShapeDtypeStruct((N_NODES, HIDDEN), jnp.float32),
    )(x, lin1t)


def _sc_body(xh_hbm, w_hbm, c_hbm, src_hbm, dst_hbm, out_hbm, cnt_hbm,
             acc, cnt, src_v, dst_v, cv, xj, wv, msg, ones_v, zbuf, zcnt,
             sem_i, sem_g, sem_w):
    c = lax.axis_index("c")
    s = lax.axis_index("s")
    wid = c * NS + s

    @pl.loop(0, 128)
    def _zb(j):
        for k in range(8):
            zbuf[j, pl.ds(k * 16, 16)] = jnp.zeros((16,), jnp.float32)

    @pl.loop(0, ROWS_PER_TILE // 16)
    def _zc(j):
        zcnt[pl.ds(j * 16, 16)] = jnp.zeros((16,), jnp.float32)

    @pl.loop(0, EB // 16)
    def _on(j):
        ones_v[pl.ds(j * 16, 16)] = jnp.ones((16,), jnp.float32)

    for b in range(ROWS_PER_TILE // 128):
        pltpu.sync_copy(zbuf, acc.at[pl.ds(s * ROWS_PER_TILE + b * 128, 128)])
    pltpu.sync_copy(zcnt, cnt.at[pl.ds(s * ROWS_PER_TILE, ROWS_PER_TILE)])

    plsc.subcore_barrier()

    base = wid * EDGES_PER_W

    @pl.loop(0, NBLK)
    def _blk(i):
        off = base + i * EB
        i1 = pltpu.async_copy(src_hbm.at[pl.ds(off, EB)], src_v, sem_i)
        i2 = pltpu.async_copy(dst_hbm.at[pl.ds(off, EB)], dst_v, sem_i)
        i3 = pltpu.async_copy(c_hbm.at[pl.ds(off, EB)],
                              cv.at[pl.ds(0, EB)], sem_i)
        wcp = pltpu.async_copy(w_hbm.at[pl.ds(off, EB)], wv, sem_w)
        i1.wait()
        gcp = pltpu.async_copy(xh_hbm.at[src_v], xj, sem_g)
        i2.wait()
        i3.wait()
        gcp.wait()
        wcp.wait()

        @pl.loop(0, EB)
        def _mul(j):
            cj = cv[pl.ds(j, 16)][0]
            for k in range(8):
                sl = pl.ds(k * 16, 16)
                msg[j, sl] = xj[j, sl] * wv[j, sl] * cj

        pltpu.sync_copy(msg, acc.at[dst_v], add=True)
        pltpu.sync_copy(ones_v, cnt.at[dst_v], add=True)

    plsc.subcore_barrier()
    pltpu.sync_copy(acc.at[pl.ds(s * ROWS_PER_TILE, ROWS_PER_TILE)],
                    out_hbm.at[c, pl.ds(s * ROWS_PER_TILE, ROWS_PER_TILE)])
    pltpu.sync_copy(cnt.at[pl.ds(s * ROWS_PER_TILE, ROWS_PER_TILE)],
                    cnt_hbm.at[c, pl.ds(s * ROWS_PER_TILE, ROWS_PER_TILE)])


def _sc_msg(xh, w, cearr, src, dst):
    mesh = plsc.VectorSubcoreMesh(core_axis_name="c", subcore_axis_name="s",
                                  num_cores=NC, num_subcores=NS)
    fn = pl.kernel(
        _sc_body,
        out_type=[
            jax.ShapeDtypeStruct((NC, N_PAD, HIDDEN), jnp.float32),
            jax.ShapeDtypeStruct((NC, N_PAD), jnp.float32),
        ],
        mesh=mesh,
        scratch_types=[
            pltpu.VMEM_SHARED((N_PAD, HIDDEN), jnp.float32),
            pltpu.VMEM_SHARED((N_PAD,), jnp.float32),
            pltpu.VMEM((EB,), jnp.int32),
            pltpu.VMEM((EB,), jnp.int32),
            pltpu.VMEM((EB + 16,), jnp.float32),
            pltpu.VMEM((EB, HIDDEN), jnp.float32),
            pltpu.VMEM((EB, HIDDEN), jnp.float32),
            pltpu.VMEM((EB, HIDDEN), jnp.float32),
            pltpu.VMEM((EB,), jnp.float32),
            pltpu.VMEM((128, HIDDEN), jnp.float32),
            pltpu.VMEM((ROWS_PER_TILE,), jnp.float32),
            pltpu.SemaphoreType.DMA,
            pltpu.SemaphoreType.DMA,
            pltpu.SemaphoreType.DMA,
        ],
    )
    return fn(xh, w, cearr, src, dst)


def _tail_body(agg_ref, cnt_ref, l2t_ref, l2b_ref, lt_ref, lb_ref, o_ref):
    ssum = agg_ref[0] + agg_ref[1]
    cnt = cnt_ref[...]
    denom = jnp.where(cnt > 0, cnt, 1.0)
    agg = ssum / denom
    t = jnp.dot(agg, l2t_ref[...], preferred_element_type=jnp.float32)
    t = t + l2b_ref[...]
    t = _silu(t)
    o = jnp.dot(t, lt_ref[...], preferred_element_type=jnp.float32)
    o_ref[...] = o + lb_ref[...]


def _tail(agg2, cnt1, lin2t, lin2_b2, lint, lin_b2):
    BN = 2000
    grid = N_NODES // BN
    return pl.pallas_call(
        _tail_body,
        grid=(grid,),
        in_specs=[
            pl.BlockSpec((NC, BN, HIDDEN), lambda i: (0, i, 0)),
            pl.BlockSpec((BN, 1), lambda i: (i, 0)),
            pl.BlockSpec((HIDDEN, HIDDEN), lambda i: (0, 0)),
            pl.BlockSpec((1, HIDDEN), lambda i: (0, 0)),
            pl.BlockSpec((HIDDEN, HIDDEN), lambda i: (0, 0)),
            pl.BlockSpec((1, HIDDEN), lambda i: (0, 0)),
        ],
        out_specs=pl.BlockSpec((BN, HIDDEN), lambda i: (i, 0)),
        out_shape=jax.ShapeDtypeStruct((N_NODES, HIDDEN), jnp.float32),
    )(agg2, cnt1, lin2t, lin2_b2, lint, lin_b2)


def kernel(x, edge_index, edge_weight, edge_attr,
           mlp_w1, mlp_b1, mlp_w2, mlp_b2,
           lin1_w, lin2_w, lin2_b, lin_w, lin_b):
    src = edge_index[0].astype(jnp.int32)
    dst = edge_index[1].astype(jnp.int32)

    eat = edge_attr.T
    w1t = mlp_w1.T
    w2t = mlp_w2.T
    lin1t = lin1_w.T
    lin2t = lin2_w.T
    lint = lin_w.T
    b1 = mlp_b1.reshape(1, HIDDEN)
    b2 = mlp_b2.reshape(1, HIDDEN)
    lin2_b2 = lin2_b.reshape(1, HIDDEN)
    lin_b2 = lin_b.reshape(1, HIDDEN)

    w = _filter_w(eat, w1t, b1, w2t, b2)
    cearr = _cutoff(edge_weight)
    xh = _xh(x, lin1t)

    agg2, cnt2 = _sc_msg(xh, w, cearr, src, dst)

    cnt1 = (cnt2[0] + cnt2[1])[:N_NODES].reshape(N_NODES, 1)
    return _tail(agg2, cnt1, lin2t, lin2_b2, lint, lin_b2)
